# Initial kernel scaffold; baseline (speedup 1.0000x reference)
import jax
import jax.numpy as jnp
from jax.experimental import pallas as pl


def kernel(x, edge_index, W, b):
    raise NotImplementedError("write your pallas kernel here")


# speedup vs baseline: 11.1396x; 11.1396x over previous
import functools

import jax
import jax.numpy as jnp
from jax import lax
from jax.experimental import pallas as pl
from jax.experimental.pallas import tpu as pltpu
from jax.experimental.pallas import tpu_sc as plsc

N = 10000
D = 128
H = 64
E = 320000
LANES = 128
EROWS = 2560
EPAD = EROWS * LANES - E
NC, NS = 2, 16
RA = EROWS // (NC * NS)
RC = EROWS // NS
DEGR = 80
NAGG = 10112
SLICE = NAGG // NS
DEGS = 8

_MESH = dict(core_axis_name="c", subcore_axis_name="s", num_cores=NC,
             num_subcores=NS)


def _deg_body(dst_hbm, deg_hbm, dstv, part1, part, rowidx, zb, sdeg):
    c = lax.axis_index("c")
    s = lax.axis_index("s")
    w = c * NS + s
    z16 = jnp.zeros((16,), jnp.float32)
    ones = jnp.ones((16,), jnp.float32)

    def z1(i, _):
        part1[pl.ds(i * 16, 16)] = z16
        return 0
    lax.fori_loop(0, DEGR * LANES // 16, z1, 0)

    pltpu.sync_copy(dst_hbm.at[pl.ds(w * RA, RA)], dstv)

    def hr(r, _):
        def hk(k, _):
            idx = dstv[r, pl.ds(k * 16, 16)]
            plsc.addupdate_scatter(part1, [idx], ones)
            return 0
        return lax.fori_loop(0, LANES // 16, hk, 0)
    lax.fori_loop(0, RA, hr, 0)

    def cr(r, _):
        def ck(k, _):
            part[r, pl.ds(k * 16, 16)] = part1[pl.ds(r * LANES + k * 16, 16)]
            return 0
        return lax.fori_loop(0, LANES // 16, ck, 0)
    lax.fori_loop(0, DEGR, cr, 0)

    def zz(i, _):
        def zzk(k, _):
            zb[i, pl.ds(k * 16, 16)] = z16
            return 0
        return lax.fori_loop(0, LANES // 16, zzk, 0)
    lax.fori_loop(0, DEGS, zz, 0)

    @pl.when(s < DEGR // DEGS)
    def _():
        pltpu.sync_copy(zb, sdeg.at[pl.ds(s * DEGS, DEGS)])

    iota = lax.iota(jnp.int32, 16)

    def ri(i, _):
        rowidx[pl.ds(i * 16, 16)] = iota + i * 16
        return 0
    lax.fori_loop(0, DEGR // 16, ri, 0)

    plsc.subcore_barrier()
    pltpu.sync_copy(part, sdeg.at[rowidx], add=True)
    plsc.subcore_barrier()

    @pl.when(s < DEGR // DEGS)
    def _():
        pltpu.sync_copy(sdeg.at[pl.ds(s * DEGS, DEGS)], zb)
        pltpu.sync_copy(zb, deg_hbm.at[c, pl.ds(s * DEGS, DEGS)])


_deg_call = pl.kernel(
    _deg_body,
    out_type=jax.ShapeDtypeStruct((NC, DEGR, LANES), jnp.float32),
    mesh=plsc.VectorSubcoreMesh(**_MESH),
    scratch_types=[
        pltpu.VMEM((RA, LANES), jnp.int32),
        pltpu.VMEM((DEGR * LANES,), jnp.float32),
        pltpu.VMEM((DEGR, LANES), jnp.float32),
        pltpu.VMEM((DEGR,), jnp.int32),
        pltpu.VMEM((DEGS, LANES), jnp.float32),
        pltpu.VMEM_SHARED((DEGR, LANES), jnp.float32),
    ],
    compiler_params=pltpu.CompilerParams(needs_layout_passes=False),
)


def _agg_body(hhat_hbm, src_hbm, dst_hbm, agg_hbm, srcv, dstv, buf, zb,
              agg_sp):
    c = lax.axis_index("c")
    s = lax.axis_index("s")
    w = c * NS + s
    z16 = jnp.zeros((16,), jnp.float32)

    def zr(r, _):
        def zk(k, _):
            zb[r, pl.ds(k * 16, 16)] = z16
            return 0
        return lax.fori_loop(0, LANES // 16, zk, 0)
    lax.fori_loop(0, DEGS, zr, 0)

    def zs(i, _):
        pltpu.sync_copy(zb, agg_sp.at[pl.ds(s * SLICE + i * DEGS, DEGS)])
        return 0
    lax.fori_loop(0, SLICE // DEGS, zs, 0)

    pltpu.sync_copy(src_hbm.at[pl.ds(w * RA, RA)], srcv)
    pltpu.sync_copy(dst_hbm.at[pl.ds(w * RA, RA)], dstv)

    plsc.subcore_barrier()

    def step(j, _):
        pltpu.sync_copy(hhat_hbm.at[srcv.at[j]], buf)
        pltpu.sync_copy(buf, agg_sp.at[dstv.at[j]], add=True)
        return 0
    lax.fori_loop(0, RA, step, 0)

    plsc.subcore_barrier()

    pltpu.sync_copy(agg_sp.at[pl.ds(s * SLICE, SLICE)],
                    agg_hbm.at[c, pl.ds(s * SLICE, SLICE)])


_agg_call = pl.kernel(
    _agg_body,
    out_type=jax.ShapeDtypeStruct((NC, NAGG, D), jnp.float32),
    mesh=plsc.VectorSubcoreMesh(**_MESH),
    scratch_types=[
        pltpu.VMEM((RA, LANES), jnp.int32),
        pltpu.VMEM((RA, LANES), jnp.int32),
        pltpu.VMEM((LANES, D), jnp.float32),
        pltpu.VMEM((DEGS, D), jnp.float32),
        pltpu.VMEM_SHARED((NAGG, D), jnp.float32),
    ],
    compiler_params=pltpu.CompilerParams(
        needs_layout_passes=False, use_tc_tiling_on_sc=False),
)


def _mm_body(x_ref, w_ref, da_ref, db_ref, out_ref):
    deg = da_ref[...] + db_ref[...] + 1.0
    dinv = lax.rsqrt(deg)
    h = jnp.dot(x_ref[...], w_ref[...], preferred_element_type=jnp.float32)
    out_ref[...] = h * dinv


def _mm_call(x, w, dega, degb):
    blk = 400
    return pl.pallas_call(
        _mm_body,
        grid=(N // blk,),
        in_specs=[
            pl.BlockSpec((blk, D), lambda i: (i, 0)),
            pl.BlockSpec((D, D), lambda i: (0, 0)),
            pl.BlockSpec((blk, 1), lambda i: (i, 0)),
            pl.BlockSpec((blk, 1), lambda i: (i, 0)),
        ],
        out_specs=pl.BlockSpec((blk, D), lambda i: (i, 0)),
        out_shape=jax.ShapeDtypeStruct((N, D), jnp.float32),
    )(x, w, dega, degb)


def _ep_body(agg_ref, hh_ref, da_ref, db_ref, b_ref, out_ref):
    deg = da_ref[...] + db_ref[...] + 1.0
    dinv = lax.rsqrt(deg)
    v = agg_ref[0] + agg_ref[1] + hh_ref[...]
    out_ref[...] = jnp.maximum(v * dinv + b_ref[...], 0.0)


def _ep_call(agg, hhat, dega, degb, b2):
    blk = 512
    grid = (N + blk - 1) // blk
    return pl.pallas_call(
        _ep_body,
        grid=(grid,),
        in_specs=[
            pl.BlockSpec((NC, blk, D), lambda i: (0, i, 0)),
            pl.BlockSpec((blk, D), lambda i: (i, 0)),
            pl.BlockSpec((blk, 1), lambda i: (i, 0)),
            pl.BlockSpec((blk, 1), lambda i: (i, 0)),
            pl.BlockSpec((1, D), lambda i: (0, 0)),
        ],
        out_specs=pl.BlockSpec((blk, D), lambda i: (i, 0)),
        out_shape=jax.ShapeDtypeStruct((N, D), jnp.float32),
    )(agg, hhat, dega, degb, b2)


def kernel(x, edge_index, W, b):
    src = edge_index[0].astype(jnp.int32)
    dst = edge_index[1].astype(jnp.int32)
    srcp = jnp.concatenate(
        [src, jnp.zeros((EPAD,), jnp.int32)]).reshape(EROWS, LANES)
    dstp = jnp.concatenate(
        [dst, jnp.full((EPAD,), N, jnp.int32)]).reshape(EROWS, LANES)

    deg2 = _deg_call(dstp).reshape(NC, DEGR * LANES)
    dega = deg2[0, :N].reshape(N, 1)
    degb = deg2[1, :N].reshape(N, 1)

    hhat = _mm_call(x, W, dega, degb)
    agg = _agg_call(hhat, srcp, dstp)
    return _ep_call(agg, hhat, dega, degb, b.reshape(1, D))


# speedup vs baseline: 11.3830x; 1.0219x over previous
import functools

import jax
import jax.numpy as jnp
from jax import lax
from jax.experimental import pallas as pl
from jax.experimental.pallas import tpu as pltpu
from jax.experimental.pallas import tpu_sc as plsc

N = 10000
D = 128
H = 64
E = 320000
LANES = 128
CW = 64
EROWS = 5120
EPAD = EROWS * CW - E
NC, NS = 2, 16
RA = EROWS // (NC * NS)
DEGR = 80
NAGG = 10112
SLICE = NAGG // NS
DEGS = 8

_MESH = dict(core_axis_name="c", subcore_axis_name="s", num_cores=NC,
             num_subcores=NS)


def _deg_body(dst_hbm, deg_hbm, dstv, part1, part, rowidx, zb, sdeg):
    c = lax.axis_index("c")
    s = lax.axis_index("s")
    w = c * NS + s
    z16 = jnp.zeros((16,), jnp.float32)
    ones = jnp.ones((16,), jnp.float32)

    def z1(i, _):
        part1[pl.ds(i * 16, 16)] = z16
        return 0
    lax.fori_loop(0, DEGR * LANES // 16, z1, 0)

    pltpu.sync_copy(dst_hbm.at[pl.ds(w * RA, RA)], dstv)

    def hr(r, _):
        def hk(k, _):
            idx = dstv[r, pl.ds(k * 16, 16)]
            plsc.addupdate_scatter(part1, [idx], ones)
            return 0
        return lax.fori_loop(0, CW // 16, hk, 0)
    lax.fori_loop(0, RA, hr, 0)

    def cr(r, _):
        def ck(k, _):
            part[r, pl.ds(k * 16, 16)] = part1[pl.ds(r * LANES + k * 16, 16)]
            return 0
        return lax.fori_loop(0, LANES // 16, ck, 0)
    lax.fori_loop(0, DEGR, cr, 0)

    def zz(i, _):
        def zzk(k, _):
            zb[i, pl.ds(k * 16, 16)] = z16
            return 0
        return lax.fori_loop(0, LANES // 16, zzk, 0)
    lax.fori_loop(0, DEGS, zz, 0)

    @pl.when(s < DEGR // DEGS)
    def _():
        pltpu.sync_copy(zb, sdeg.at[pl.ds(s * DEGS, DEGS)])

    iota = lax.iota(jnp.int32, 16)

    def ri(i, _):
        rowidx[pl.ds(i * 16, 16)] = iota + i * 16
        return 0
    lax.fori_loop(0, DEGR // 16, ri, 0)

    plsc.subcore_barrier()
    pltpu.sync_copy(part, sdeg.at[rowidx], add=True)
    plsc.subcore_barrier()

    @pl.when(s < DEGR // DEGS)
    def _():
        pltpu.sync_copy(sdeg.at[pl.ds(s * DEGS, DEGS)], zb)
        pltpu.sync_copy(zb, deg_hbm.at[c, pl.ds(s * DEGS, DEGS)])


_deg_call = pl.kernel(
    _deg_body,
    out_type=jax.ShapeDtypeStruct((NC, DEGR, LANES), jnp.float32),
    mesh=plsc.VectorSubcoreMesh(**_MESH),
    scratch_types=[
        pltpu.VMEM((RA, CW), jnp.int32),
        pltpu.VMEM((DEGR * LANES,), jnp.float32),
        pltpu.VMEM((DEGR, LANES), jnp.float32),
        pltpu.VMEM((DEGR,), jnp.int32),
        pltpu.VMEM((DEGS, LANES), jnp.float32),
        pltpu.VMEM_SHARED((DEGR, LANES), jnp.float32),
    ],
    compiler_params=pltpu.CompilerParams(needs_layout_passes=False),
)


def _agg_body(hhat_hbm, src_hbm, dst_hbm, agg_hbm, srcv, dstv, buf_a, buf_b,
              zb, gsa, gsb, ssa, ssb, agg_sp):
    c = lax.axis_index("c")
    s = lax.axis_index("s")
    w = c * NS + s
    z16 = jnp.zeros((16,), jnp.float32)

    def zr(r, _):
        def zk(k, _):
            zb[r, pl.ds(k * 16, 16)] = z16
            return 0
        return lax.fori_loop(0, LANES // 16, zk, 0)
    lax.fori_loop(0, DEGS, zr, 0)

    def zs(i, _):
        pltpu.sync_copy(zb, agg_sp.at[pl.ds(s * SLICE + i * DEGS, DEGS)])
        return 0
    lax.fori_loop(0, SLICE // DEGS, zs, 0)

    pltpu.sync_copy(src_hbm.at[pl.ds(w * RA, RA)], srcv)
    pltpu.sync_copy(dst_hbm.at[pl.ds(w * RA, RA)], dstv)

    plsc.subcore_barrier()

    def g_issue(j, buf, sem):
        pltpu.async_copy(hhat_hbm.at[srcv.at[j]], buf, sem)

    def g_wait(buf, sem):
        pltpu.make_async_copy(hhat_hbm.at[srcv.at[0]], buf, sem).wait()

    def s_issue(j, buf, sem):
        pltpu.async_copy(buf, agg_sp.at[dstv.at[j]], sem, add=True)

    def s_wait(buf, sem):
        pltpu.make_async_copy(buf, agg_sp.at[dstv.at[0]], sem).wait()

    g_issue(0, buf_a, gsa)
    g_wait(buf_a, gsa)
    s_issue(0, buf_a, ssa)
    g_issue(1, buf_b, gsb)

    def step(i, _):
        j = 2 * i + 1
        g_wait(buf_b, gsb)
        s_issue(j, buf_b, ssb)
        s_wait(buf_a, ssa)
        g_issue(j + 1, buf_a, gsa)
        g_wait(buf_a, gsa)
        s_issue(j + 1, buf_a, ssa)
        s_wait(buf_b, ssb)
        g_issue(j + 2, buf_b, gsb)
        return 0
    lax.fori_loop(0, RA // 2 - 1, step, 0)

    g_wait(buf_b, gsb)
    s_issue(RA - 1, buf_b, ssb)
    s_wait(buf_a, ssa)
    s_wait(buf_b, ssb)

    plsc.subcore_barrier()

    pltpu.sync_copy(agg_sp.at[pl.ds(s * SLICE, SLICE)],
                    agg_hbm.at[c, pl.ds(s * SLICE, SLICE)])


_agg_call = pl.kernel(
    _agg_body,
    out_type=jax.ShapeDtypeStruct((NC, NAGG, D), jnp.float32),
    mesh=plsc.VectorSubcoreMesh(**_MESH),
    scratch_types=[
        pltpu.VMEM((RA, CW), jnp.int32),
        pltpu.VMEM((RA, CW), jnp.int32),
        pltpu.VMEM((CW, D), jnp.float32),
        pltpu.VMEM((CW, D), jnp.float32),
        pltpu.VMEM((DEGS, D), jnp.float32),
        pltpu.SemaphoreType.DMA,
        pltpu.SemaphoreType.DMA,
        pltpu.SemaphoreType.DMA,
        pltpu.SemaphoreType.DMA,
        pltpu.VMEM_SHARED((NAGG, D), jnp.float32),
    ],
    compiler_params=pltpu.CompilerParams(
        needs_layout_passes=False, use_tc_tiling_on_sc=False),
)


def _mm_body(x_ref, w_ref, da_ref, db_ref, out_ref):
    deg = da_ref[...] + db_ref[...] + 1.0
    dinv = lax.rsqrt(deg)
    h = jnp.dot(x_ref[...], w_ref[...], preferred_element_type=jnp.float32)
    out_ref[...] = h * dinv


def _mm_call(x, w, dega, degb):
    blk = 400
    return pl.pallas_call(
        _mm_body,
        grid=(N // blk,),
        in_specs=[
            pl.BlockSpec((blk, D), lambda i: (i, 0)),
            pl.BlockSpec((D, D), lambda i: (0, 0)),
            pl.BlockSpec((blk, 1), lambda i: (i, 0)),
            pl.BlockSpec((blk, 1), lambda i: (i, 0)),
        ],
        out_specs=pl.BlockSpec((blk, D), lambda i: (i, 0)),
        out_shape=jax.ShapeDtypeStruct((N, D), jnp.float32),
    )(x, w, dega, degb)


def _ep_body(agg_ref, hh_ref, da_ref, db_ref, b_ref, out_ref):
    deg = da_ref[...] + db_ref[...] + 1.0
    dinv = lax.rsqrt(deg)
    v = agg_ref[0] + agg_ref[1] + hh_ref[...]
    out_ref[...] = jnp.maximum(v * dinv + b_ref[...], 0.0)


def _ep_call(agg, hhat, dega, degb, b2):
    blk = 512
    grid = (N + blk - 1) // blk
    return pl.pallas_call(
        _ep_body,
        grid=(grid,),
        in_specs=[
            pl.BlockSpec((NC, blk, D), lambda i: (0, i, 0)),
            pl.BlockSpec((blk, D), lambda i: (i, 0)),
            pl.BlockSpec((blk, 1), lambda i: (i, 0)),
            pl.BlockSpec((blk, 1), lambda i: (i, 0)),
            pl.BlockSpec((1, D), lambda i: (0, 0)),
        ],
        out_specs=pl.BlockSpec((blk, D), lambda i: (i, 0)),
        out_shape=jax.ShapeDtypeStruct((N, D), jnp.float32),
    )(agg, hhat, dega, degb, b2)


def kernel(x, edge_index, W, b):
    src = edge_index[0].astype(jnp.int32)
    dst = edge_index[1].astype(jnp.int32)
    srcp = jnp.concatenate(
        [src, jnp.zeros((EPAD,), jnp.int32)]).reshape(EROWS, CW)
    dstp = jnp.concatenate(
        [dst, jnp.full((EPAD,), N, jnp.int32)]).reshape(EROWS, CW)

    deg2 = _deg_call(dstp).reshape(NC, DEGR * LANES)
    dega = deg2[0, :N].reshape(N, 1)
    degb = deg2[1, :N].reshape(N, 1)

    hhat = _mm_call(x, W, dega, degb)
    agg = _agg_call(hhat, srcp, dstp)
    return _ep_call(agg, hhat, dega, degb, b.reshape(1, D))


# speedup vs baseline: 28.8833x; 2.5374x over previous
import functools

import jax
import jax.numpy as jnp
from jax import lax
from jax.experimental import pallas as pl
from jax.experimental.pallas import tpu as pltpu
from jax.experimental.pallas import tpu_sc as plsc

N = 10000
D = 128
H = 64
E = 320000
LANES = 128
CW = 64
EROWS = 5120
EPAD = EROWS * CW - E
NC, NS = 2, 16
RA = EROWS // (NC * NS)
DEGR = 80
NAGG = 10112
SLICE = NAGG // NS
DEGS = 8

_MESH = dict(core_axis_name="c", subcore_axis_name="s", num_cores=NC,
             num_subcores=NS)


def _deg_body(dst_hbm, deg_hbm, dstv, part1, part, rowidx, zb, sdeg):
    c = lax.axis_index("c")
    s = lax.axis_index("s")
    w = c * NS + s
    z16 = jnp.zeros((16,), jnp.float32)
    ones = jnp.ones((16,), jnp.float32)

    def z1(i, _):
        part1[pl.ds(i * 16, 16)] = z16
        return 0
    lax.fori_loop(0, DEGR * LANES // 16, z1, 0)

    pltpu.sync_copy(dst_hbm.at[pl.ds(w * RA, RA)], dstv)

    def hr(r, _):
        def hk(k, _):
            idx = dstv[r, pl.ds(k * 16, 16)]
            plsc.addupdate_scatter(part1, [idx], ones)
            return 0
        return lax.fori_loop(0, CW // 16, hk, 0)
    lax.fori_loop(0, RA, hr, 0)

    def cr(r, _):
        def ck(k, _):
            part[r, pl.ds(k * 16, 16)] = part1[pl.ds(r * LANES + k * 16, 16)]
            return 0
        return lax.fori_loop(0, LANES // 16, ck, 0)
    lax.fori_loop(0, DEGR, cr, 0)

    def zz(i, _):
        def zzk(k, _):
            zb[i, pl.ds(k * 16, 16)] = z16
            return 0
        return lax.fori_loop(0, LANES // 16, zzk, 0)
    lax.fori_loop(0, DEGS, zz, 0)

    @pl.when(s < DEGR // DEGS)
    def _():
        pltpu.sync_copy(zb, sdeg.at[pl.ds(s * DEGS, DEGS)])

    iota = lax.iota(jnp.int32, 16)

    def ri(i, _):
        rowidx[pl.ds(i * 16, 16)] = iota + i * 16
        return 0
    lax.fori_loop(0, DEGR // 16, ri, 0)

    plsc.subcore_barrier()
    pltpu.sync_copy(part, sdeg.at[rowidx], add=True)
    plsc.subcore_barrier()

    @pl.when(s < DEGR // DEGS)
    def _():
        pltpu.sync_copy(sdeg.at[pl.ds(s * DEGS, DEGS)], zb)
        pltpu.sync_copy(zb, deg_hbm.at[c, pl.ds(s * DEGS, DEGS)])


_deg_call = pl.kernel(
    _deg_body,
    out_type=jax.ShapeDtypeStruct((NC, DEGR, LANES), jnp.float32),
    mesh=plsc.VectorSubcoreMesh(**_MESH),
    scratch_types=[
        pltpu.VMEM((RA, CW), jnp.int32),
        pltpu.VMEM((DEGR * LANES,), jnp.float32),
        pltpu.VMEM((DEGR, LANES), jnp.float32),
        pltpu.VMEM((DEGR,), jnp.int32),
        pltpu.VMEM((DEGS, LANES), jnp.float32),
        pltpu.VMEM_SHARED((DEGR, LANES), jnp.float32),
    ],
    compiler_params=pltpu.CompilerParams(needs_layout_passes=False),
)


def _agg_body(hhat_hbm, src_hbm, dst_hbm, agg_hbm, srcv, dstv, buf_a, buf_b,
              zb, gsa, gsb, ssa, ssb, agg_sp):
    c = lax.axis_index("c")
    s = lax.axis_index("s")
    w = c * NS + s
    z16 = jnp.zeros((16,), jnp.float32)

    def zr(r, _):
        def zk(k, _):
            zb[r, pl.ds(k * 16, 16)] = z16
            return 0
        return lax.fori_loop(0, LANES // 16, zk, 0)
    lax.fori_loop(0, DEGS, zr, 0)

    def zs(i, _):
        pltpu.sync_copy(zb, agg_sp.at[pl.ds(s * SLICE + i * DEGS, DEGS)])
        return 0
    lax.fori_loop(0, SLICE // DEGS, zs, 0)

    pltpu.sync_copy(src_hbm.at[pl.ds(w * RA, RA)], srcv)
    pltpu.sync_copy(dst_hbm.at[pl.ds(w * RA, RA)], dstv)

    plsc.subcore_barrier()

    def g_issue(j, buf, sem):
        pltpu.async_copy(hhat_hbm.at[srcv.at[j]], buf, sem)

    def g_wait(buf, sem):
        pltpu.make_async_copy(hhat_hbm.at[srcv.at[0]], buf, sem).wait()

    def s_issue(j, buf, sem):
        pltpu.async_copy(buf, agg_sp.at[dstv.at[j]], sem, add=True)

    def s_wait(buf, sem):
        pltpu.make_async_copy(buf, agg_sp.at[dstv.at[0]], sem).wait()

    g_issue(0, buf_a, gsa)
    g_wait(buf_a, gsa)
    s_issue(0, buf_a, ssa)
    g_issue(1, buf_b, gsb)

    def step(i, _):
        j = 2 * i + 1
        g_wait(buf_b, gsb)
        s_issue(j, buf_b, ssb)
        s_wait(buf_a, ssa)
        g_issue(j + 1, buf_a, gsa)
        g_wait(buf_a, gsa)
        s_issue(j + 1, buf_a, ssa)
        s_wait(buf_b, ssb)
        g_issue(j + 2, buf_b, gsb)
        return 0
    lax.fori_loop(0, RA // 2 - 1, step, 0)

    g_wait(buf_b, gsb)
    s_issue(RA - 1, buf_b, ssb)
    s_wait(buf_a, ssa)
    s_wait(buf_b, ssb)

    plsc.subcore_barrier()

    pltpu.sync_copy(agg_sp.at[pl.ds(s * SLICE, SLICE)],
                    agg_hbm.at[c, pl.ds(s * SLICE, SLICE)])


_agg_call = pl.kernel(
    _agg_body,
    out_type=jax.ShapeDtypeStruct((NC, NAGG, D), jnp.float32),
    mesh=plsc.VectorSubcoreMesh(**_MESH),
    scratch_types=[
        pltpu.VMEM((RA, CW), jnp.int32),
        pltpu.VMEM((RA, CW), jnp.int32),
        pltpu.VMEM((CW, D), jnp.float32),
        pltpu.VMEM((CW, D), jnp.float32),
        pltpu.VMEM((DEGS, D), jnp.float32),
        pltpu.SemaphoreType.DMA,
        pltpu.SemaphoreType.DMA,
        pltpu.SemaphoreType.DMA,
        pltpu.SemaphoreType.DMA,
        pltpu.VMEM_SHARED((NAGG, D), jnp.float32),
    ],
    compiler_params=pltpu.CompilerParams(
        needs_layout_passes=False, use_tc_tiling_on_sc=False),
)


def _mm_body(x_ref, w_ref, da_ref, db_ref, out_ref):
    deg = da_ref[...] + db_ref[...] + 1.0
    dinv = lax.rsqrt(deg)
    h = jnp.dot(x_ref[...], w_ref[...], preferred_element_type=jnp.float32)
    out_ref[...] = h * dinv


def _mm_call(x, w, dega, degb):
    blk = 400
    return pl.pallas_call(
        _mm_body,
        grid=(N // blk,),
        in_specs=[
            pl.BlockSpec((blk, D), lambda i: (i, 0)),
            pl.BlockSpec((D, D), lambda i: (0, 0)),
            pl.BlockSpec((blk, 1), lambda i: (i, 0)),
            pl.BlockSpec((blk, 1), lambda i: (i, 0)),
        ],
        out_specs=pl.BlockSpec((blk, D), lambda i: (i, 0)),
        out_shape=jax.ShapeDtypeStruct((N, D), jnp.float32),
    )(x, w, dega, degb)


def _ep_body(agg_ref, hh_ref, da_ref, db_ref, b_ref, out_ref):
    deg = da_ref[...] + db_ref[...] + 1.0
    dinv = lax.rsqrt(deg)
    v = agg_ref[0] + agg_ref[1] + hh_ref[...]
    out_ref[...] = jnp.maximum(v * dinv + b_ref[...], 0.0)


def _ep_call(agg, hhat, dega, degb, b2):
    blk = 512
    grid = (N + blk - 1) // blk
    return pl.pallas_call(
        _ep_body,
        grid=(grid,),
        in_specs=[
            pl.BlockSpec((NC, blk, D), lambda i: (0, i, 0)),
            pl.BlockSpec((blk, D), lambda i: (i, 0)),
            pl.BlockSpec((blk, 1), lambda i: (i, 0)),
            pl.BlockSpec((blk, 1), lambda i: (i, 0)),
            pl.BlockSpec((1, D), lambda i: (0, 0)),
        ],
        out_specs=pl.BlockSpec((blk, D), lambda i: (i, 0)),
        out_shape=jax.ShapeDtypeStruct((N, D), jnp.float32),
    )(agg, hhat, dega, degb, b2)


def kernel(x, edge_index, W, b):
    src = edge_index[0].astype(jnp.int32)
    dst = edge_index[1].astype(jnp.int32)
    cyc = jnp.arange(EPAD, dtype=jnp.int32) % CW
    srcp = jnp.concatenate([src, cyc]).reshape(EROWS, CW)
    dstp = jnp.concatenate([dst, N + cyc]).reshape(EROWS, CW)

    deg2 = _deg_call(dstp).reshape(NC, DEGR * LANES)
    dega = deg2[0, :N].reshape(N, 1)
    degb = deg2[1, :N].reshape(N, 1)

    hhat = _mm_call(x, W, dega, degb)
    agg = _agg_call(hhat, srcp, dstp)
    return _ep_call(agg, hhat, dega, degb, b.reshape(1, D))


# speedup vs baseline: 39.0144x; 1.3508x over previous
import functools

import jax
import jax.numpy as jnp
from jax import lax
from jax.experimental import pallas as pl
from jax.experimental.pallas import tpu as pltpu
from jax.experimental.pallas import tpu_sc as plsc

N = 10000
D = 128
H = 64
E = 320000
LANES = 128
CW = 64
EROWS = 5120
EPAD = EROWS * CW - E
NC, NS = 2, 16
RA = EROWS // (NC * NS)
DEGR = 80
NAGG = 10112
SLICE = NAGG // NS
DEGS = 8

_MESH = dict(core_axis_name="c", subcore_axis_name="s", num_cores=NC,
             num_subcores=NS)


def _deg_body(dst_hbm, deg_hbm, dstv, part1, part, rowidx, zb, sdeg):
    c = lax.axis_index("c")
    s = lax.axis_index("s")
    w = c * NS + s
    z16 = jnp.zeros((16,), jnp.float32)
    ones = jnp.ones((16,), jnp.float32)

    def z1(i, _):
        part1[pl.ds(i * 16, 16)] = z16
        return 0
    lax.fori_loop(0, DEGR * LANES // 16, z1, 0)

    pltpu.sync_copy(dst_hbm.at[pl.ds(w * RA, RA)], dstv)

    def hr(r, _):
        def hk(k, _):
            idx = dstv[r, pl.ds(k * 16, 16)]
            plsc.addupdate_scatter(part1, [idx], ones)
            return 0
        return lax.fori_loop(0, CW // 16, hk, 0)
    lax.fori_loop(0, RA, hr, 0)

    def cr(r, _):
        def ck(k, _):
            part[r, pl.ds(k * 16, 16)] = part1[pl.ds(r * LANES + k * 16, 16)]
            return 0
        return lax.fori_loop(0, LANES // 16, ck, 0)
    lax.fori_loop(0, DEGR, cr, 0)

    def zz(i, _):
        def zzk(k, _):
            zb[i, pl.ds(k * 16, 16)] = z16
            return 0
        return lax.fori_loop(0, LANES // 16, zzk, 0)
    lax.fori_loop(0, DEGS, zz, 0)

    @pl.when(s < DEGR // DEGS)
    def _():
        pltpu.sync_copy(zb, sdeg.at[pl.ds(s * DEGS, DEGS)])

    iota = lax.iota(jnp.int32, 16)

    def ri(i, _):
        rowidx[pl.ds(i * 16, 16)] = iota + i * 16
        return 0
    lax.fori_loop(0, DEGR // 16, ri, 0)

    plsc.subcore_barrier()
    pltpu.sync_copy(part, sdeg.at[rowidx], add=True)
    plsc.subcore_barrier()

    @pl.when(s < DEGR // DEGS)
    def _():
        pltpu.sync_copy(sdeg.at[pl.ds(s * DEGS, DEGS)], zb)
        pltpu.sync_copy(zb, deg_hbm.at[c, pl.ds(s * DEGS, DEGS)])


_deg_call = pl.kernel(
    _deg_body,
    out_type=jax.ShapeDtypeStruct((NC, DEGR, LANES), jnp.float32),
    mesh=plsc.VectorSubcoreMesh(**_MESH),
    scratch_types=[
        pltpu.VMEM((RA, CW), jnp.int32),
        pltpu.VMEM((DEGR * LANES,), jnp.float32),
        pltpu.VMEM((DEGR, LANES), jnp.float32),
        pltpu.VMEM((DEGR,), jnp.int32),
        pltpu.VMEM((DEGS, LANES), jnp.float32),
        pltpu.VMEM_SHARED((DEGR, LANES), jnp.float32),
    ],
    compiler_params=pltpu.CompilerParams(needs_layout_passes=False),
)


def _agg_body(hhat_hbm, src_hbm, dst_hbm, agg_hbm, srcv, dstv, buf_a, buf_b,
              buf_c, zb, gsa, gsb, gsc, ssa, ssb, ssc, agg_sp):
    c = lax.axis_index("c")
    s = lax.axis_index("s")
    w = c * NS + s
    z16 = jnp.zeros((16,), jnp.float32)

    def zr(r, _):
        def zk(k, _):
            zb[r, pl.ds(k * 16, 16)] = z16
            return 0
        return lax.fori_loop(0, LANES // 16, zk, 0)
    lax.fori_loop(0, DEGS, zr, 0)

    def zs(i, _):
        pltpu.sync_copy(zb, agg_sp.at[pl.ds(s * SLICE + i * DEGS, DEGS)])
        return 0
    lax.fori_loop(0, SLICE // DEGS, zs, 0)

    pltpu.sync_copy(src_hbm.at[pl.ds(w * RA, RA)], srcv)
    pltpu.sync_copy(dst_hbm.at[pl.ds(w * RA, RA)], dstv)

    plsc.subcore_barrier()

    def g_issue(j, buf, sem):
        pltpu.async_copy(hhat_hbm.at[srcv.at[j]], buf, sem)

    def g_wait(buf, sem):
        pltpu.make_async_copy(hhat_hbm.at[srcv.at[0]], buf, sem).wait()

    def s_issue(j, buf, sem):
        pltpu.async_copy(buf, agg_sp.at[dstv.at[j]], sem, add=True)

    def s_wait(buf, sem):
        pltpu.make_async_copy(buf, agg_sp.at[dstv.at[0]], sem).wait()

    def zc(r, _):
        def zk(k, _):
            buf_c[r, pl.ds(k * 16, 16)] = z16
            return 0
        return lax.fori_loop(0, D // 16, zk, 0)
    lax.fori_loop(0, CW, zc, 0)
    g_issue(0, buf_a, gsa)
    g_issue(1, buf_b, gsb)
    s_issue(0, buf_c, ssc)

    def step(i, _):
        c = 3 * i
        g_wait(buf_a, gsa)
        s_issue(c, buf_a, ssa)
        s_wait(buf_c, ssc)
        g_issue(c + 2, buf_c, gsc)
        g_wait(buf_b, gsb)
        s_issue(c + 1, buf_b, ssb)
        s_wait(buf_a, ssa)
        g_issue(c + 3, buf_a, gsa)
        g_wait(buf_c, gsc)
        s_issue(c + 2, buf_c, ssc)
        s_wait(buf_b, ssb)
        g_issue(jnp.minimum(c + 4, RA - 1), buf_b, gsb)
        return 0
    lax.fori_loop(0, RA // 3, step, 0)

    g_wait(buf_a, gsa)
    s_issue(RA - 1, buf_a, ssa)
    g_wait(buf_b, gsb)
    s_wait(buf_c, ssc)
    s_wait(buf_a, ssa)

    plsc.subcore_barrier()

    pltpu.sync_copy(agg_sp.at[pl.ds(s * SLICE, SLICE)],
                    agg_hbm.at[c, pl.ds(s * SLICE, SLICE)])


_agg_call = pl.kernel(
    _agg_body,
    out_type=jax.ShapeDtypeStruct((NC, NAGG, D), jnp.float32),
    mesh=plsc.VectorSubcoreMesh(**_MESH),
    scratch_types=[
        pltpu.VMEM((RA, CW), jnp.int32),
        pltpu.VMEM((RA, CW), jnp.int32),
        pltpu.VMEM((CW, D), jnp.float32),
        pltpu.VMEM((CW, D), jnp.float32),
        pltpu.VMEM((CW, D), jnp.float32),
        pltpu.VMEM((DEGS, D), jnp.float32),
        pltpu.SemaphoreType.DMA,
        pltpu.SemaphoreType.DMA,
        pltpu.SemaphoreType.DMA,
        pltpu.SemaphoreType.DMA,
        pltpu.SemaphoreType.DMA,
        pltpu.SemaphoreType.DMA,
        pltpu.VMEM_SHARED((NAGG, D), jnp.float32),
    ],
    compiler_params=pltpu.CompilerParams(
        needs_layout_passes=False, use_tc_tiling_on_sc=False),
)


def _mm_body(x_ref, w_ref, da_ref, db_ref, out_ref):
    deg = da_ref[...] + db_ref[...] + 1.0
    dinv = lax.rsqrt(deg)
    h = jnp.dot(x_ref[...], w_ref[...], preferred_element_type=jnp.float32)
    out_ref[...] = h * dinv


def _mm_call(x, w, dega, degb):
    blk = 400
    return pl.pallas_call(
        _mm_body,
        grid=(N // blk,),
        in_specs=[
            pl.BlockSpec((blk, D), lambda i: (i, 0)),
            pl.BlockSpec((D, D), lambda i: (0, 0)),
            pl.BlockSpec((blk, 1), lambda i: (i, 0)),
            pl.BlockSpec((blk, 1), lambda i: (i, 0)),
        ],
        out_specs=pl.BlockSpec((blk, D), lambda i: (i, 0)),
        out_shape=jax.ShapeDtypeStruct((N, D), jnp.float32),
    )(x, w, dega, degb)


def _ep_body(agg_ref, hh_ref, da_ref, db_ref, b_ref, out_ref):
    deg = da_ref[...] + db_ref[...] + 1.0
    dinv = lax.rsqrt(deg)
    v = agg_ref[0] + agg_ref[1] + hh_ref[...]
    out_ref[...] = jnp.maximum(v * dinv + b_ref[...], 0.0)


def _ep_call(agg, hhat, dega, degb, b2):
    blk = 512
    grid = (N + blk - 1) // blk
    return pl.pallas_call(
        _ep_body,
        grid=(grid,),
        in_specs=[
            pl.BlockSpec((NC, blk, D), lambda i: (0, i, 0)),
            pl.BlockSpec((blk, D), lambda i: (i, 0)),
            pl.BlockSpec((blk, 1), lambda i: (i, 0)),
            pl.BlockSpec((blk, 1), lambda i: (i, 0)),
            pl.BlockSpec((1, D), lambda i: (0, 0)),
        ],
        out_specs=pl.BlockSpec((blk, D), lambda i: (i, 0)),
        out_shape=jax.ShapeDtypeStruct((N, D), jnp.float32),
    )(agg, hhat, dega, degb, b2)


def kernel(x, edge_index, W, b):
    src = edge_index[0].astype(jnp.int32)
    dst = edge_index[1].astype(jnp.int32)
    cyc = jnp.arange(EPAD, dtype=jnp.int32) % CW
    srcp = jnp.concatenate([src, cyc]).reshape(EROWS, CW)
    dstp = jnp.concatenate([dst, N + cyc]).reshape(EROWS, CW)

    deg2 = _deg_call(dstp).reshape(NC, DEGR * LANES)
    dega = deg2[0, :N].reshape(N, 1)
    degb = deg2[1, :N].reshape(N, 1)

    hhat = _mm_call(x, W, dega, degb)
    agg = _agg_call(hhat, srcp, dstp)
    return _ep_call(agg, hhat, dega, degb, b.reshape(1, D))


# speedup vs baseline: 55.0623x; 1.4113x over previous
import functools

import jax
import jax.numpy as jnp
from jax import lax
from jax.experimental import pallas as pl
from jax.experimental.pallas import tpu as pltpu
from jax.experimental.pallas import tpu_sc as plsc

N = 10000
D = 128
E = 320000
LANES = 128
CW = 128
EROWS = 2560
NC, NS = 2, 16
RA = EROWS // (NC * NS)
DEGR = 80
NAGG = 10240
SLICE = NAGG // NS
DEGS = 8
BR = 1024

_MESH = dict(core_axis_name="c", subcore_axis_name="s", num_cores=NC,
             num_subcores=NS)


def _deg_body(dst_hbm, deg_hbm, dstv, part1, part, rowidx, zb, sdeg):
    c = lax.axis_index("c")
    s = lax.axis_index("s")
    w = c * NS + s
    z16 = jnp.zeros((16,), jnp.float32)
    ones = jnp.ones((16,), jnp.float32)

    def z1(i, _):
        part1[pl.ds(i * 16, 16)] = z16
        return 0
    lax.fori_loop(0, DEGR * LANES // 16, z1, 0)

    pltpu.sync_copy(dst_hbm.at[pl.ds(w * RA, RA)], dstv)

    def hr(r, _):
        def hk(k, _):
            idx = dstv[r, pl.ds(k * 16, 16)]
            plsc.addupdate_scatter(part1, [idx], ones)
            return 0
        return lax.fori_loop(0, CW // 16, hk, 0)
    lax.fori_loop(0, RA, hr, 0)

    def cr(r, _):
        def ck(k, _):
            part[r, pl.ds(k * 16, 16)] = part1[pl.ds(r * LANES + k * 16, 16)]
            return 0
        return lax.fori_loop(0, LANES // 16, ck, 0)
    lax.fori_loop(0, DEGR, cr, 0)

    def zz(i, _):
        def zzk(k, _):
            zb[i, pl.ds(k * 16, 16)] = z16
            return 0
        return lax.fori_loop(0, LANES // 16, zzk, 0)
    lax.fori_loop(0, DEGS, zz, 0)

    @pl.when(s < DEGR // DEGS)
    def _():
        pltpu.sync_copy(zb, sdeg.at[pl.ds(s * DEGS, DEGS)])

    iota = lax.iota(jnp.int32, 16)

    def ri(i, _):
        rowidx[pl.ds(i * 16, 16)] = iota + i * 16
        return 0
    lax.fori_loop(0, DEGR // 16, ri, 0)

    plsc.subcore_barrier()
    pltpu.sync_copy(part, sdeg.at[rowidx], add=True)
    plsc.subcore_barrier()

    @pl.when(s < DEGR // DEGS)
    def _():
        pltpu.sync_copy(sdeg.at[pl.ds(s * DEGS, DEGS)], zb)
        pltpu.sync_copy(zb, deg_hbm.at[c, pl.ds(s * DEGS, DEGS)])


_deg_call = pl.kernel(
    _deg_body,
    out_type=jax.ShapeDtypeStruct((NC, DEGR, LANES), jnp.float32),
    mesh=plsc.VectorSubcoreMesh(**_MESH),
    scratch_types=[
        pltpu.VMEM((RA, CW), jnp.int32),
        pltpu.VMEM((DEGR * LANES,), jnp.float32),
        pltpu.VMEM((DEGR, LANES), jnp.float32),
        pltpu.VMEM((DEGR,), jnp.int32),
        pltpu.VMEM((DEGS, LANES), jnp.float32),
        pltpu.VMEM_SHARED((DEGR, LANES), jnp.float32),
    ],
    compiler_params=pltpu.CompilerParams(
        needs_layout_passes=False, use_tc_tiling_on_sc=False),
)


def _agg_body(hhat_hbm, src_hbm, dst_hbm, agg_hbm, srcv, dstv, buf_a, buf_b,
              buf_c, zb, gsa, gsb, gsc, ssa, ssb, ssc, agg_sp):
    c = lax.axis_index("c")
    s = lax.axis_index("s")
    w = c * NS + s
    z32 = jnp.zeros((32,), jnp.bfloat16)

    def zr(r, _):
        def zk(k, _):
            zb[r, pl.ds(k * 32, 32)] = z32
            return 0
        return lax.fori_loop(0, LANES // 32, zk, 0)
    lax.fori_loop(0, DEGS, zr, 0)

    def zs(i, _):
        pltpu.sync_copy(zb, agg_sp.at[pl.ds(s * SLICE + i * DEGS, DEGS)])
        return 0
    lax.fori_loop(0, SLICE // DEGS, zs, 0)

    pltpu.sync_copy(src_hbm.at[pl.ds(w * RA, RA)], srcv)
    pltpu.sync_copy(dst_hbm.at[pl.ds(w * RA, RA)], dstv)

    plsc.subcore_barrier()

    def g_issue(j, buf, sem):
        pltpu.async_copy(hhat_hbm.at[srcv.at[j]], buf, sem)

    def g_wait(buf, sem):
        pltpu.make_async_copy(hhat_hbm.at[srcv.at[0]], buf, sem).wait()

    def s_issue(j, buf, sem):
        pltpu.async_copy(buf, agg_sp.at[dstv.at[j]], sem, add=True)

    def s_wait(buf, sem):
        pltpu.make_async_copy(buf, agg_sp.at[dstv.at[0]], sem).wait()

    def zc(r, _):
        def zk(k, _):
            buf_c[r, pl.ds(k * 32, 32)] = z32
            return 0
        return lax.fori_loop(0, D // 32, zk, 0)
    lax.fori_loop(0, CW, zc, 0)
    g_issue(0, buf_a, gsa)
    g_issue(1, buf_b, gsb)
    s_issue(0, buf_c, ssc)

    def step(i, _):
        c = 3 * i
        g_wait(buf_a, gsa)
        s_issue(c, buf_a, ssa)
        s_wait(buf_c, ssc)
        g_issue(c + 2, buf_c, gsc)
        g_wait(buf_b, gsb)
        s_issue(c + 1, buf_b, ssb)
        s_wait(buf_a, ssa)
        g_issue(c + 3, buf_a, gsa)
        g_wait(buf_c, gsc)
        s_issue(c + 2, buf_c, ssc)
        s_wait(buf_b, ssb)
        g_issue(jnp.minimum(c + 4, RA - 1), buf_b, gsb)
        return 0
    lax.fori_loop(0, RA // 3, step, 0)

    g_wait(buf_a, gsa)
    s_issue(RA - 2, buf_a, ssa)
    g_wait(buf_b, gsb)
    s_issue(RA - 1, buf_b, ssb)
    s_wait(buf_c, ssc)
    s_wait(buf_a, ssa)
    s_wait(buf_b, ssb)

    plsc.subcore_barrier()

    pltpu.sync_copy(agg_sp.at[pl.ds(s * SLICE, SLICE)],
                    agg_hbm.at[c, pl.ds(s * SLICE, SLICE)])


_agg_call = pl.kernel(
    _agg_body,
    out_type=jax.ShapeDtypeStruct((NC, NAGG, D), jnp.bfloat16),
    mesh=plsc.VectorSubcoreMesh(**_MESH),
    scratch_types=[
        pltpu.VMEM((RA, CW), jnp.int32),
        pltpu.VMEM((RA, CW), jnp.int32),
        pltpu.VMEM((CW, D), jnp.bfloat16),
        pltpu.VMEM((CW, D), jnp.bfloat16),
        pltpu.VMEM((CW, D), jnp.bfloat16),
        pltpu.VMEM((DEGS, D), jnp.bfloat16),
        pltpu.SemaphoreType.DMA,
        pltpu.SemaphoreType.DMA,
        pltpu.SemaphoreType.DMA,
        pltpu.SemaphoreType.DMA,
        pltpu.SemaphoreType.DMA,
        pltpu.SemaphoreType.DMA,
        pltpu.VMEM_SHARED((NAGG, D), jnp.bfloat16),
    ],
    compiler_params=pltpu.CompilerParams(
        needs_layout_passes=False, use_tc_tiling_on_sc=False),
)


def _slice_body(ei_ref, srcp_ref, dstp_ref):
    v = ei_ref[...]
    er = E // CW
    srcp_ref[pl.ds(0, er), :] = jnp.reshape(v[0], (er, CW))
    dstp_ref[pl.ds(0, er), :] = jnp.reshape(v[1], (er, CW))
    cyc = lax.broadcasted_iota(jnp.int32, (EROWS - er, CW), 1)
    srcp_ref[pl.ds(er, EROWS - er), :] = cyc
    dstp_ref[pl.ds(er, EROWS - er), :] = cyc + N


def _slice_call(ei):
    return pl.pallas_call(
        _slice_body,
        out_shape=[jax.ShapeDtypeStruct((EROWS, CW), jnp.int32),
                   jax.ShapeDtypeStruct((EROWS, CW), jnp.int32)],
    )(ei)


def _mm_body(x_ref, w_ref, dg_ref, out_ref):
    dinv = lax.rsqrt(dg_ref[0] + dg_ref[1] + 1.0)
    h = jnp.dot(x_ref[...], w_ref[...], preferred_element_type=jnp.float32)
    h3 = jnp.reshape(h, (BR // LANES, LANES, D))
    out_ref[...] = jnp.reshape(h3 * dinv[:, :, None], (BR, D)).astype(
        jnp.bfloat16)


def _mm_call(x, w, deg2):
    return pl.pallas_call(
        _mm_body,
        grid=(DEGR * LANES // BR,),
        in_specs=[
            pl.BlockSpec((BR, D), lambda i: (i, 0)),
            pl.BlockSpec((D, D), lambda i: (0, 0)),
            pl.BlockSpec((NC, BR // LANES, LANES), lambda i: (0, i, 0)),
        ],
        out_specs=pl.BlockSpec((BR, D), lambda i: (i, 0)),
        out_shape=jax.ShapeDtypeStruct((N, D), jnp.bfloat16),
    )(x, w, deg2)


def _ep_body(agg_ref, hh_ref, dg_ref, b_ref, out_ref):
    dinv = lax.rsqrt(dg_ref[0] + dg_ref[1] + 1.0)
    v = (agg_ref[0].astype(jnp.float32) + agg_ref[1].astype(jnp.float32)
         + hh_ref[...].astype(jnp.float32))
    v3 = jnp.reshape(v, (BR // LANES, LANES, D))
    v3 = v3 * dinv[:, :, None] + b_ref[0][None, None, :]
    out_ref[...] = jnp.maximum(jnp.reshape(v3, (BR, D)), 0.0)


def _ep_call(agg, hhat, deg2, b2):
    return pl.pallas_call(
        _ep_body,
        grid=(DEGR * LANES // BR,),
        in_specs=[
            pl.BlockSpec((NC, BR, D), lambda i: (0, i, 0)),
            pl.BlockSpec((BR, D), lambda i: (i, 0)),
            pl.BlockSpec((NC, BR // LANES, LANES), lambda i: (0, i, 0)),
            pl.BlockSpec((1, D), lambda i: (0, 0)),
        ],
        out_specs=pl.BlockSpec((BR, D), lambda i: (i, 0)),
        out_shape=jax.ShapeDtypeStruct((N, D), jnp.float32),
    )(agg, hhat, deg2, b2)


def kernel(x, edge_index, W, b):
    srcp, dstp = _slice_call(edge_index.astype(jnp.int32))
    deg2 = _deg_call(dstp)
    hhat = _mm_call(x, W, deg2)
    agg = _agg_call(hhat, srcp, dstp)
    return _ep_call(agg, hhat, deg2, b.reshape(1, D))
